# baseline (speedup 1.0000x reference)
import functools

import jax
import jax.numpy as jnp
from jax import lax
from jax.experimental import pallas as pl
from jax.experimental.pallas import tpu as pltpu
from jax.experimental.pallas import tpu_sc as plsc

_N = 262144
_EPS = 1e-7
_ROWS = _N // 128
_DROWS = 4 * _ROWS

_BLK = 512
_STEPS = _ROWS // _BLK
_C = 32
_NCH = _BLK // _C


def _cls_kernel(ts_ref, os_ref, out_ref, bce_ref, val_ref, pos_ref):
    i = pl.program_id(0)

    bce_acc = jnp.zeros((_C, 128), jnp.float32)
    val_acc = jnp.zeros((_C, 128), jnp.float32)
    pos_acc = jnp.zeros((_C, 128), jnp.float32)

    for k in range(_NCH):
        ts = ts_ref[k * _C:(k + 1) * _C, :]
        osc = os_ref[k * _C:(k + 1) * _C, :]
        valid = (ts != -1.0).astype(jnp.float32)
        pos = ts > 0.0
        o = jnp.clip(osc, _EPS, 1.0 - _EPS)
        bce = -jnp.log(jnp.where(pos, o, 1.0 - o))
        bce_acc += bce * valid
        val_acc += valid
        pos_acc += pos.astype(jnp.float32)

    @pl.when(i == 0)
    def _init():
        bce_ref[...] = bce_acc
        val_ref[...] = val_acc
        pos_ref[...] = pos_acc

    @pl.when(i > 0)
    def _accum():
        bce_ref[...] += bce_acc
        val_ref[...] += val_acc
        pos_ref[...] += pos_acc

    @pl.when(i == _STEPS - 1)
    def _finalize():
        out_ref[0, 0] = jnp.sum(bce_ref[...]) / jnp.maximum(jnp.sum(val_ref[...]), 1.0)
        out_ref[0, 1] = jnp.sum(pos_ref[...])


def _cls_and_count(ts2d, os2d):
    return pl.pallas_call(
        _cls_kernel,
        grid=(_STEPS,),
        in_specs=[
            pl.BlockSpec((_BLK, 128), lambda i: (i, 0)),
            pl.BlockSpec((_BLK, 128), lambda i: (i, 0)),
        ],
        out_specs=pl.BlockSpec((1, 2), lambda i: (0, 0), memory_space=pltpu.SMEM),
        out_shape=jax.ShapeDtypeStruct((1, 2), jnp.float32),
        scratch_shapes=[
            pltpu.VMEM((_C, 128), jnp.float32),
            pltpu.VMEM((_C, 128), jnp.float32),
            pltpu.VMEM((_C, 128), jnp.float32),
        ],
        compiler_params=pltpu.CompilerParams(
            dimension_semantics=("arbitrary",),
        ),
    )(ts2d, os2d)


_NW = 32
_QB_PER_W = _ROWS // _NW
_TS_SPAN = _QB_PER_W * 128
_D_SPAN = _QB_PER_W * 512


def _sc_reg_body(ts_hbm, td_hbm, od_hbm, out_hbm, ts_v, td_v, od_v, acc_v):
    wid = lax.axis_index("s") * 2 + lax.axis_index("c")
    a0 = wid * _TS_SPAN
    e0 = wid * _D_SPAN
    pltpu.sync_copy(ts_hbm.at[pl.ds(a0, _TS_SPAN)], ts_v)
    pltpu.sync_copy(td_hbm.at[pl.ds(e0, _D_SPAN)], td_v)
    pltpu.sync_copy(od_hbm.at[pl.ds(e0, _D_SPAN)], od_v)

    def body(qb, acc):
        ts_off = qb * 128
        d_off = qb * 512
        for v in range(8):
            p_star = jnp.maximum(jnp.sign(ts_v[pl.ds(ts_off + 16 * v, 16)]), 0.0)
            for c in range(4):
                o = d_off + 128 * c + 16 * v
                d = jnp.abs(od_v[pl.ds(o, 16)] - td_v[pl.ds(o, 16)])
                m = jnp.minimum(d, 1.0)
                acc = acc + p_star * (m * (d - 0.5 * m))
        return acc

    acc = lax.fori_loop(0, _QB_PER_W, body, jnp.zeros((16,), jnp.float32))
    acc_v[...] = acc
    pltpu.sync_copy(acc_v, out_hbm.at[wid])


@functools.partial(
    pl.kernel,
    mesh=plsc.VectorSubcoreMesh(core_axis_name="c", subcore_axis_name="s"),
    out_type=jax.ShapeDtypeStruct((_NW, 16), jnp.float32),
    scratch_types=[
        pltpu.VMEM((_TS_SPAN,), jnp.float32),
        pltpu.VMEM((_D_SPAN,), jnp.float32),
        pltpu.VMEM((_D_SPAN,), jnp.float32),
        pltpu.VMEM((16,), jnp.float32),
    ],
)
def _sc_reg_sum(ts_hbm, td_hbm, od_hbm, out_hbm, ts_v, td_v, od_v, acc_v):
    _sc_reg_body(ts_hbm, td_hbm, od_hbm, out_hbm, ts_v, td_v, od_v, acc_v)


def kernel(target_deltas, target_scores, output_deltas, output_scores):
    ts2d = target_scores.reshape(_ROWS, 128)
    os2d = output_scores.reshape(_ROWS, 128)
    ts1d = target_scores.reshape(_N)
    td1d = jnp.transpose(target_deltas.reshape(_ROWS, 128, 4), (0, 2, 1)).reshape(4 * _N)
    od1d = jnp.transpose(output_deltas.reshape(_ROWS, 128, 4), (0, 2, 1)).reshape(4 * _N)

    reg_parts = _sc_reg_sum(ts1d, td1d, od1d)
    cls_out = _cls_and_count(ts2d, os2d)

    cls_loss = cls_out[0, 0]
    p_cnt = cls_out[0, 1]
    reg_loss = 10.0 * jnp.sum(reg_parts) / jnp.maximum(_EPS, p_cnt)
    return cls_loss + reg_loss


# speedup vs baseline: 1.0472x; 1.0472x over previous
import functools

import jax
import jax.numpy as jnp
from jax import lax
from jax.experimental import pallas as pl
from jax.experimental.pallas import tpu as pltpu
from jax.experimental.pallas import tpu_sc as plsc

_N = 262144
_EPS = 1e-7
_ROWS = _N // 128
_DROWS = 4 * _ROWS

_BLK = 512
_STEPS = _ROWS // _BLK
_C = 32
_NCH = _BLK // _C


def _cls_kernel(ts_ref, os_ref, out_ref, bce_ref, val_ref, pos_ref):
    i = pl.program_id(0)

    bce_acc = jnp.zeros((_C, 128), jnp.float32)
    val_acc = jnp.zeros((_C, 128), jnp.float32)
    pos_acc = jnp.zeros((_C, 128), jnp.float32)

    for k in range(_NCH):
        ts = ts_ref[k * _C:(k + 1) * _C, :]
        osc = os_ref[k * _C:(k + 1) * _C, :]
        valid = (ts != -1.0).astype(jnp.float32)
        pos = ts > 0.0
        o = jnp.clip(osc, _EPS, 1.0 - _EPS)
        bce = -jnp.log(jnp.where(pos, o, 1.0 - o))
        bce_acc += bce * valid
        val_acc += valid
        pos_acc += pos.astype(jnp.float32)

    @pl.when(i == 0)
    def _init():
        bce_ref[...] = bce_acc
        val_ref[...] = val_acc
        pos_ref[...] = pos_acc

    @pl.when(i > 0)
    def _accum():
        bce_ref[...] += bce_acc
        val_ref[...] += val_acc
        pos_ref[...] += pos_acc

    @pl.when(i == _STEPS - 1)
    def _finalize():
        out_ref[0, 0] = jnp.sum(bce_ref[...]) / jnp.maximum(jnp.sum(val_ref[...]), 1.0)
        out_ref[0, 1] = jnp.sum(pos_ref[...])


def _cls_and_count(ts2d, os2d):
    return pl.pallas_call(
        _cls_kernel,
        grid=(_STEPS,),
        in_specs=[
            pl.BlockSpec((_BLK, 128), lambda i: (i, 0)),
            pl.BlockSpec((_BLK, 128), lambda i: (i, 0)),
        ],
        out_specs=pl.BlockSpec((1, 2), lambda i: (0, 0), memory_space=pltpu.SMEM),
        out_shape=jax.ShapeDtypeStruct((1, 2), jnp.float32),
        scratch_shapes=[
            pltpu.VMEM((_C, 128), jnp.float32),
            pltpu.VMEM((_C, 128), jnp.float32),
            pltpu.VMEM((_C, 128), jnp.float32),
        ],
        compiler_params=pltpu.CompilerParams(
            dimension_semantics=("arbitrary",),
        ),
    )(ts2d, os2d)


_NW = 32
_QB_PER_W = _ROWS // _NW
_TS_SPAN = _QB_PER_W * 128
_D_SPAN = _QB_PER_W * 512


def _sc_reg_body(ts_hbm, td_hbm, od_hbm, out_hbm, ts_v, td_v, od_v, acc_v):
    wid = lax.axis_index("s") * 2 + lax.axis_index("c")
    a0 = wid * _TS_SPAN
    e0 = wid * _D_SPAN
    pltpu.sync_copy(ts_hbm.at[pl.ds(a0, _TS_SPAN)], ts_v)
    pltpu.sync_copy(td_hbm.at[pl.ds(e0, _D_SPAN)], td_v)
    pltpu.sync_copy(od_hbm.at[pl.ds(e0, _D_SPAN)], od_v)

    def body(qb, accs):
        ts_off = qb * 128
        d_off = qb * 512
        new = []
        for v in range(8):
            p_star = jnp.maximum(jnp.sign(ts_v[pl.ds(ts_off + 16 * v, 16)]), 0.0)
            s = None
            for c in range(4):
                o = d_off + 128 * c + 16 * v
                d = jnp.abs(od_v[pl.ds(o, 16)] - td_v[pl.ds(o, 16)])
                m = jnp.minimum(d, 1.0)
                sl1 = m * (d - 0.5 * m)
                s = sl1 if s is None else s + sl1
            new.append(accs[v] + p_star * s)
        return tuple(new)

    accs = lax.fori_loop(
        0, _QB_PER_W, body,
        tuple(jnp.zeros((16,), jnp.float32) for _ in range(8)))
    acc = accs[0]
    for v in range(1, 8):
        acc = acc + accs[v]
    acc_v[...] = acc
    pltpu.sync_copy(acc_v, out_hbm.at[wid])


@functools.partial(
    pl.kernel,
    mesh=plsc.VectorSubcoreMesh(core_axis_name="c", subcore_axis_name="s"),
    out_type=jax.ShapeDtypeStruct((_NW, 16), jnp.float32),
    scratch_types=[
        pltpu.VMEM((_TS_SPAN,), jnp.float32),
        pltpu.VMEM((_D_SPAN,), jnp.float32),
        pltpu.VMEM((_D_SPAN,), jnp.float32),
        pltpu.VMEM((16,), jnp.float32),
    ],
)
def _sc_reg_sum(ts_hbm, td_hbm, od_hbm, out_hbm, ts_v, td_v, od_v, acc_v):
    _sc_reg_body(ts_hbm, td_hbm, od_hbm, out_hbm, ts_v, td_v, od_v, acc_v)


def kernel(target_deltas, target_scores, output_deltas, output_scores):
    ts2d = target_scores.reshape(_ROWS, 128)
    os2d = output_scores.reshape(_ROWS, 128)
    ts1d = target_scores.reshape(_N)
    td1d = jnp.transpose(target_deltas.reshape(_ROWS, 128, 4), (0, 2, 1)).reshape(4 * _N)
    od1d = jnp.transpose(output_deltas.reshape(_ROWS, 128, 4), (0, 2, 1)).reshape(4 * _N)

    reg_parts = _sc_reg_sum(ts1d, td1d, od1d)
    cls_out = _cls_and_count(ts2d, os2d)

    cls_loss = cls_out[0, 0]
    p_cnt = cls_out[0, 1]
    reg_loss = 10.0 * jnp.sum(reg_parts) / jnp.maximum(_EPS, p_cnt)
    return cls_loss + reg_loss


# speedup vs baseline: 5.5502x; 5.2999x over previous
import jax
import jax.numpy as jnp
from jax.experimental import pallas as pl
from jax.experimental.pallas import tpu as pltpu

_N = 262144
_EPS = 1e-7
_ROWS = _N // 128
_DROWS = 4 * _ROWS
_BLK = 1024
_DBLK = 4 * _BLK
_STEPS = _ROWS // _BLK
_C = 32
_NCH = _BLK // _C


def _rpn_loss_kernel(ts_ref, os_ref, td_ref, od_ref, out_ref,
                     bce_ref, val_ref, reg_ref, pos_ref):
    i = pl.program_id(0)

    bce_acc = jnp.zeros((_C, 128), jnp.float32)
    val_acc = jnp.zeros((_C, 128), jnp.float32)
    reg_acc = jnp.zeros((_C, 128), jnp.float32)
    pos_acc = jnp.zeros((_C, 128), jnp.float32)

    for k in range(_NCH):
        ts = ts_ref[k * _C:(k + 1) * _C, :]
        osc = os_ref[k * _C:(k + 1) * _C, :]
        valid = (ts != -1.0).astype(jnp.float32)
        pos = ts > 0.0
        p_star = pos.astype(jnp.float32)
        o = jnp.clip(osc, _EPS, 1.0 - _EPS)
        bce = -jnp.log(jnp.where(pos, o, 1.0 - o))
        bce_acc += bce * valid
        val_acc += valid
        pos_acc += p_star

        for j in range(4):
            r0 = 4 * k * _C + j * _C
            q0 = k * _C + j * (_C // 4)
            mask = jnp.broadcast_to(
                (ts_ref[q0:q0 + _C // 4, :] > 0.0).astype(jnp.float32)[:, None, :],
                (_C // 4, 4, 128)).reshape(_C, 128)
            d = jnp.abs(od_ref[r0:r0 + _C, :] - td_ref[r0:r0 + _C, :])
            m = jnp.minimum(d, 1.0)
            reg_acc += (m * (d - 0.5 * m)) * mask

    @pl.when(i == 0)
    def _init():
        bce_ref[...] = bce_acc
        val_ref[...] = val_acc
        reg_ref[...] = reg_acc
        pos_ref[...] = pos_acc

    @pl.when(i > 0)
    def _accum():
        bce_ref[...] += bce_acc
        val_ref[...] += val_acc
        reg_ref[...] += reg_acc
        pos_ref[...] += pos_acc

    @pl.when(i == _STEPS - 1)
    def _finalize():
        cls_loss = jnp.sum(bce_ref[...]) / jnp.maximum(jnp.sum(val_ref[...]), 1.0)
        reg_loss = 10.0 * jnp.sum(reg_ref[...]) / jnp.maximum(_EPS, jnp.sum(pos_ref[...]))
        out_ref[0, 0] = cls_loss + reg_loss


def kernel(target_deltas, target_scores, output_deltas, output_scores):
    ts = target_scores.reshape(_ROWS, 128)
    osc = output_scores.reshape(_ROWS, 128)
    td = jnp.transpose(target_deltas.reshape(_ROWS, 128, 4), (0, 2, 1)).reshape(_DROWS, 128)
    od = jnp.transpose(output_deltas.reshape(_ROWS, 128, 4), (0, 2, 1)).reshape(_DROWS, 128)

    out = pl.pallas_call(
        _rpn_loss_kernel,
        grid=(_STEPS,),
        in_specs=[
            pl.BlockSpec((_BLK, 128), lambda i: (i, 0)),
            pl.BlockSpec((_BLK, 128), lambda i: (i, 0)),
            pl.BlockSpec((_DBLK, 128), lambda i: (i, 0)),
            pl.BlockSpec((_DBLK, 128), lambda i: (i, 0)),
        ],
        out_specs=pl.BlockSpec((1, 1), lambda i: (0, 0), memory_space=pltpu.SMEM),
        out_shape=jax.ShapeDtypeStruct((1, 1), jnp.float32),
        scratch_shapes=[
            pltpu.VMEM((_C, 128), jnp.float32),
            pltpu.VMEM((_C, 128), jnp.float32),
            pltpu.VMEM((_C, 128), jnp.float32),
            pltpu.VMEM((_C, 128), jnp.float32),
        ],
        compiler_params=pltpu.CompilerParams(
            dimension_semantics=("arbitrary",),
        ),
    )(ts, osc, td, od)
    return out[0, 0]
